# baseline (speedup 1.0000x reference)
import functools

import numpy as np
import jax
import jax.numpy as jnp
from jax import lax
from jax.experimental import pallas as pl
from jax.experimental.pallas import tpu as pltpu
from jax.experimental.pallas import tpu_sc as plsc

P = 0.5

B, C, T, D = 64, 3, 128, 256
ROWS = B * C * T
NC, NS = 2, 16
NW = NC * NS
RPW = ROWS // NW
NBLK = RPW // T

_CONSTS = None


def _get_consts():
    global _CONSTS
    if _CONSTS is None:
        with jax.ensure_compile_time_eval():
            key = jax.random.key(42)
            coins, perms = [], []
            for i in range(2):
                kk = jax.random.fold_in(key, i)
                kc, kp = jax.random.split(kk)
                coins.append(bool(jax.random.uniform(kc) < P))
                perms.append(np.asarray(jax.random.permutation(kp, T)))
        eff = [p if c else np.arange(T, dtype=np.int32) for c, p in zip(coins, perms)]
        labels = np.tile(
            np.array([[float(coins[0]), float(coins[1])]], dtype=np.float32),
            (B, 1))
        _CONSTS = ([tuple(int(v) for v in e) for e in eff], labels)
    return _CONSTS


def _make_permute_rows(perm0, perm1):
    @functools.partial(
        pl.kernel,
        out_type=(
            jax.ShapeDtypeStruct((ROWS, D), jnp.float32),
            jax.ShapeDtypeStruct((ROWS, D), jnp.float32),
        ),
        mesh=plsc.VectorSubcoreMesh(core_axis_name="c", subcore_axis_name="s"),
        scratch_types=[pltpu.SemaphoreType.DMA],
    )
    def _permute_rows(audio_hbm, acc_hbm, audio_out, acc_out, sem):
        wid = lax.axis_index("s") * NC + lax.axis_index("c")
        row0 = wid * RPW

        def block(j, carry):
            base = row0 + j * T
            for k in range(T):
                pltpu.async_copy(
                    audio_hbm.at[pl.ds(base + perm0[k], 1)],
                    audio_out.at[pl.ds(base + k, 1)], sem)
                pltpu.async_copy(
                    acc_hbm.at[pl.ds(base + perm1[k], 1)],
                    acc_out.at[pl.ds(base + k, 1)], sem)
            return carry

        lax.fori_loop(0, NBLK, block, 0)
        pltpu.make_async_copy(
            audio_hbm.at[pl.ds(row0, RPW)],
            audio_out.at[pl.ds(row0, RPW)], sem).wait()
        pltpu.make_async_copy(
            acc_hbm.at[pl.ds(row0, RPW)],
            acc_out.at[pl.ds(row0, RPW)], sem).wait()

    return _permute_rows


def kernel(shake_audio, shake_acc):
    (perm0, perm1), labels_np = _get_consts()
    a2 = shake_audio.reshape(ROWS, D)
    c2 = shake_acc.reshape(ROWS, D)
    out_a, out_c = _make_permute_rows(perm0, perm1)(a2, c2)
    return (out_a.reshape(shake_audio.shape),
            out_c.reshape(shake_acc.shape),
            jnp.asarray(labels_np))


# speedup vs baseline: 26.8173x; 26.8173x over previous
import functools

import numpy as np
import jax
import jax.numpy as jnp
from jax import lax
from jax.experimental import pallas as pl
from jax.experimental.pallas import tpu as pltpu
from jax.experimental.pallas import tpu_sc as plsc

P = 0.5

B, C, T, D = 64, 3, 128, 256
ROWS = B * C * T
NC, NS = 2, 16
NW = NC * NS
RPW = ROWS // NW
CH = 128
NCHUNK = RPW // CH
NSTREAM = 2 * NCHUNK
NBUF = 3
LAG = 2

_CONSTS = None


def _get_consts():
    global _CONSTS
    if _CONSTS is None:
        with jax.ensure_compile_time_eval():
            key = jax.random.key(42)
            coins, perms = [], []
            for i in range(2):
                kk = jax.random.fold_in(key, i)
                kc, kp = jax.random.split(kk)
                coins.append(bool(jax.random.uniform(kc) < P))
                perms.append(np.asarray(jax.random.permutation(kp, T)))
        eff = [p if c else np.arange(T, dtype=np.int32) for c, p in zip(coins, perms)]
        r = np.arange(ROWS, dtype=np.int64)
        idx = np.zeros((NW, NSTREAM, CH), dtype=np.int32)
        for t in range(2):
            src = (r // T) * T + eff[t][r % T]
            idx[:, t * NCHUNK:(t + 1) * NCHUNK, :] = src.reshape(NW, NCHUNK, CH)
        labels = np.tile(
            np.array([[float(coins[0]), float(coins[1])]], dtype=np.float32),
            (B, 1))
        _CONSTS = (idx, labels)
    return _CONSTS


@functools.partial(
    pl.kernel,
    out_type=(
        jax.ShapeDtypeStruct((ROWS, D), jnp.float32),
        jax.ShapeDtypeStruct((ROWS, D), jnp.float32),
    ),
    mesh=plsc.VectorSubcoreMesh(core_axis_name="c", subcore_axis_name="s"),
    scratch_types=[
        pltpu.VMEM((NSTREAM, CH), jnp.int32),
        [pltpu.VMEM((CH, D), jnp.float32) for _ in range(NBUF)],
        [pltpu.SemaphoreType.DMA for _ in range(NBUF)],
        [pltpu.SemaphoreType.DMA for _ in range(NBUF)],
    ],
)
def _permute_rows(audio_hbm, acc_hbm, idx_hbm,
                  audio_out, acc_out,
                  idx_v, bufs, in_sems, out_sems):
    wid = lax.axis_index("s") * NC + lax.axis_index("c")
    pltpu.sync_copy(idx_hbm.at[wid], idx_v)

    srcs = (audio_hbm, acc_hbm)
    dsts = (audio_out, acc_out)

    in_h = [None] * NBUF
    out_h = [None] * NBUF
    for i in range(NSTREAM + LAG):
        if i < NSTREAM:
            b = i % NBUF
            t = i // NCHUNK
            if out_h[b] is not None:
                out_h[b].wait()
            in_h[b] = pltpu.async_copy(
                srcs[t].at[idx_v.at[i]], bufs[b], in_sems[b])
        j = i - LAG
        if 0 <= j:
            bj = j % NBUF
            tj, cj = divmod(j, NCHUNK)
            in_h[bj].wait()
            base = wid * RPW + cj * CH
            out_h[bj] = pltpu.async_copy(
                bufs[bj], dsts[tj].at[pl.ds(base, CH)], out_sems[bj])
    for b in range(NBUF):
        if out_h[b] is not None:
            out_h[b].wait()


def kernel(shake_audio, shake_acc):
    idx_np, labels_np = _get_consts()
    a2 = shake_audio.reshape(ROWS, D)
    c2 = shake_acc.reshape(ROWS, D)
    out_a, out_c = _permute_rows(a2, c2, jnp.asarray(idx_np))
    return (out_a.reshape(shake_audio.shape),
            out_c.reshape(shake_acc.shape),
            jnp.asarray(labels_np))
